# baseline (speedup 1.0000x reference)
import functools

import jax
import jax.numpy as jnp
from jax import lax
from jax.experimental import pallas as pl
from jax.experimental.pallas import tpu as pltpu
from jax.experimental.pallas import tpu_sc as plsc

_NC = 2
_NS = 16
_NW = _NC * _NS
_LANES = 16
_ICHUNK = 128


def _sc_mean_pool(ids3, table, batch, ctx, dim, ipw_pad):
    bpw = batch // _NW
    nchunk = ipw_pad // _ICHUNK
    inv = 1.0 / ctx

    mesh = plsc.VectorSubcoreMesh(core_axis_name="c", subcore_axis_name="s",
                                  num_cores=_NC, num_subcores=_NS)

    @functools.partial(
        pl.kernel,
        out_type=jax.ShapeDtypeStruct((batch, dim), jnp.float32),
        mesh=mesh,
        scratch_types=[
            pltpu.VMEM((nchunk, _ICHUNK), jnp.int32),
            pltpu.VMEM((ipw_pad, dim), jnp.float32),
            pltpu.VMEM((bpw, dim), jnp.float32),
            pltpu.SemaphoreType.DMA,
        ],
        compiler_params=pltpu.CompilerParams(use_tc_tiling_on_sc=False),
    )
    def body(ids_hbm, table_hbm, out_hbm, idx_v, rows_v, hid_v, sem):
        wid = lax.axis_index("s") * _NC + lax.axis_index("c")
        pltpu.sync_copy(ids_hbm.at[wid], idx_v)
        copies = []
        for j in range(nchunk):
            copies.append(
                pltpu.async_copy(
                    table_hbm.at[idx_v.at[j]],
                    rows_v.at[pl.ds(j * _ICHUNK, _ICHUNK)],
                    sem,
                ))
        for c in copies:
            c.wait()

        nk = dim // _LANES

        def row_body(r, _):
            rbase = r * ctx
            accs = [jnp.zeros((_LANES,), jnp.float32) for _ in range(nk)]
            for j in range(ctx):
                for k in range(nk):
                    accs[k] = accs[k] + rows_v[rbase + j,
                                               pl.ds(k * _LANES, _LANES)]
            for k in range(nk):
                hid_v[r, pl.ds(k * _LANES, _LANES)] = accs[k] * inv
            return 0

        lax.fori_loop(0, bpw, row_body, 0)
        pltpu.sync_copy(hid_v, out_hbm.at[pl.ds(wid * bpw, bpw)])

    return body(ids3, table)


def _mm_body(h_ref, w_ref, b_ref, o_ref):
    h = h_ref[...].astype(jnp.bfloat16)
    w = w_ref[...].astype(jnp.bfloat16)
    acc = lax.dot_general(h, w, (((1,), (1,)), ((), ())),
                          preferred_element_type=jnp.float32)
    o_ref[...] = acc + b_ref[...]


def _tc_matmul(hidden, W, b2, vblk):
    batch, dim = hidden.shape
    vocab = W.shape[0]
    grid = (pl.cdiv(vocab, vblk),)
    return pl.pallas_call(
        _mm_body,
        grid=grid,
        in_specs=[
            pl.BlockSpec((batch, dim), lambda i: (0, 0)),
            pl.BlockSpec((vblk, dim), lambda i: (i, 0)),
            pl.BlockSpec((1, vblk), lambda i: (0, i)),
        ],
        out_specs=pl.BlockSpec((batch, vblk), lambda i: (0, i)),
        out_shape=jax.ShapeDtypeStruct((batch, vocab), jnp.float32),
        compiler_params=pltpu.CompilerParams(
            dimension_semantics=("arbitrary",)),
    )(hidden, W, b2)


def kernel(context_ids, emb_table, W, b):
    batch, ctx = context_ids.shape
    vocab, dim = emb_table.shape

    ipw = (batch // _NW) * ctx
    ipw_pad = ((ipw + _ICHUNK - 1) // _ICHUNK) * _ICHUNK
    ids = context_ids.astype(jnp.int32).reshape(_NW, ipw)
    ids = jnp.pad(ids, ((0, 0), (0, ipw_pad - ipw)))
    ids3 = ids.reshape(_NW, ipw_pad // _ICHUNK, _ICHUNK)

    hidden = _sc_mean_pool(ids3, emb_table, batch, ctx, dim, ipw_pad)
    out = _tc_matmul(hidden, W, b.reshape(1, vocab), 1024)
    return out


# speedup vs baseline: 2.5309x; 2.5309x over previous
import functools

import jax
import jax.numpy as jnp
from jax import lax
from jax.experimental import pallas as pl
from jax.experimental.pallas import tpu as pltpu
from jax.experimental.pallas import tpu_sc as plsc

_NC = 2
_NS = 16
_NW = _NC * _NS
_LANES = 16
_ICHUNK = 128


def _sc_mean_pool(ids3, table, batch, ctx, dim, ipw_pad):
    bpw = batch // _NW
    nchunk = ipw_pad // _ICHUNK
    inv = 1.0 / ctx

    mesh = plsc.VectorSubcoreMesh(core_axis_name="c", subcore_axis_name="s",
                                  num_cores=_NC, num_subcores=_NS)

    @functools.partial(
        pl.kernel,
        out_type=jax.ShapeDtypeStruct((batch, dim), jnp.float32),
        mesh=mesh,
        scratch_types=[
            pltpu.VMEM((nchunk, _ICHUNK), jnp.int32),
            pltpu.VMEM((ipw_pad, dim), jnp.float32),
            pltpu.VMEM((bpw, dim), jnp.float32),
            pltpu.SemaphoreType.DMA,
        ],
        compiler_params=pltpu.CompilerParams(use_tc_tiling_on_sc=False),
    )
    def body(ids_hbm, table_hbm, out_hbm, idx_v, rows_v, hid_v, sem):
        wid = lax.axis_index("s") * _NC + lax.axis_index("c")
        pltpu.sync_copy(ids_hbm.at[wid], idx_v)
        copies = []
        for j in range(nchunk):
            copies.append(
                pltpu.async_copy(
                    table_hbm.at[idx_v.at[j]],
                    rows_v.at[pl.ds(j * _ICHUNK, _ICHUNK)],
                    sem,
                ))
        for c in copies:
            c.wait()

        nk = dim // _LANES

        def row_body(r, _):
            rbase = r * ctx
            accs = [jnp.zeros((_LANES,), jnp.float32) for _ in range(nk)]
            for j in range(ctx):
                for k in range(nk):
                    accs[k] = accs[k] + rows_v[rbase + j,
                                               pl.ds(k * _LANES, _LANES)]
            for k in range(nk):
                hid_v[r, pl.ds(k * _LANES, _LANES)] = accs[k] * inv
            return 0

        lax.fori_loop(0, bpw, row_body, 0)
        pltpu.sync_copy(hid_v, out_hbm.at[pl.ds(wid * bpw, bpw)])

    return body(ids3, table)


def _mm_body(wt_ref, h_ref, b_ref, o_ref):
    wt = wt_ref[...].astype(jnp.bfloat16)
    h = h_ref[...].astype(jnp.bfloat16)
    acc = lax.dot_general(wt, h, (((0,), (1,)), ((), ())),
                          preferred_element_type=jnp.float32)
    bias = jnp.transpose(b_ref[...], (1, 0))
    o_ref[...] = acc + bias


def _tc_matmul_t(hidden, Wt, b2, vblk):
    batch, dim = hidden.shape
    vocab = Wt.shape[1]
    grid = (pl.cdiv(vocab, vblk),)
    return pl.pallas_call(
        _mm_body,
        grid=grid,
        in_specs=[
            pl.BlockSpec((dim, vblk), lambda i: (0, i)),
            pl.BlockSpec((batch, dim), lambda i: (0, 0)),
            pl.BlockSpec((1, vblk), lambda i: (0, i)),
        ],
        out_specs=pl.BlockSpec((vblk, batch), lambda i: (i, 0)),
        out_shape=jax.ShapeDtypeStruct((vocab, batch), jnp.float32),
        compiler_params=pltpu.CompilerParams(
            dimension_semantics=("parallel",)),
    )(Wt, hidden, b2)


def kernel(context_ids, emb_table, W, b):
    batch, ctx = context_ids.shape
    vocab, dim = emb_table.shape

    ipw = (batch // _NW) * ctx
    ipw_pad = ((ipw + _ICHUNK - 1) // _ICHUNK) * _ICHUNK
    ids = context_ids.astype(jnp.int32).reshape(_NW, ipw)
    ids = jnp.pad(ids, ((0, 0), (0, ipw_pad - ipw)))
    ids3 = ids.reshape(_NW, ipw_pad // _ICHUNK, _ICHUNK)

    hidden = _sc_mean_pool(ids3, emb_table, batch, ctx, dim, ipw_pad)
    out_t = _tc_matmul_t(hidden, W.T, b.reshape(1, vocab), 2048)
    return out_t.T


# speedup vs baseline: 2.9948x; 1.1833x over previous
import functools

import jax
import jax.numpy as jnp
from jax import lax
from jax.experimental import pallas as pl
from jax.experimental.pallas import tpu as pltpu
from jax.experimental.pallas import tpu_sc as plsc

_NC = 2
_NS = 16
_NW = _NC * _NS
_LANES = 16
_ICHUNK = 128


def _sc_mean_pool(ids3, table, batch, ctx, dim):
    bpw = batch // _NW
    rpc = 2
    nchunk = bpw // rpc
    ipc = rpc * ctx
    inv = 1.0 / ctx
    nk = dim // _LANES

    mesh = plsc.VectorSubcoreMesh(core_axis_name="c", subcore_axis_name="s",
                                  num_cores=_NC, num_subcores=_NS)

    @functools.partial(
        pl.kernel,
        out_type=jax.ShapeDtypeStruct((batch, dim), jnp.float32),
        mesh=mesh,
        scratch_types=[
            pltpu.VMEM((nchunk, ipc), jnp.int32),
            pltpu.VMEM((bpw * ctx, dim), jnp.float32),
            pltpu.VMEM((bpw, dim), jnp.float32),
            pltpu.SemaphoreType.DMA,
        ],
        compiler_params=pltpu.CompilerParams(use_tc_tiling_on_sc=False),
    )
    def body(ids_hbm, table_hbm, out_hbm, idx_v, rows_v, hid_v, sem):
        wid = lax.axis_index("s") * _NC + lax.axis_index("c")
        pltpu.sync_copy(ids_hbm.at[wid], idx_v)
        copies = [
            pltpu.async_copy(
                table_hbm.at[idx_v.at[c]],
                rows_v.at[pl.ds(c * ipc, ipc)],
                sem,
            )
            for c in range(nchunk)
        ]
        unroll = 5
        for c in range(nchunk):
            copies[c].wait()
            for t in range(rpc):
                base = c * ipc + t * ctx

                def jbody(j, accs, base=base):
                    row = base + j * unroll
                    for u in range(unroll):
                        for k in range(nk):
                            accs = (accs[:k]
                                    + (accs[k] + rows_v[row + u,
                                                        pl.ds(k * _LANES,
                                                              _LANES)],)
                                    + accs[k + 1:])
                    return accs

                accs = tuple(
                    jnp.zeros((_LANES,), jnp.float32) for _ in range(nk))
                accs = lax.fori_loop(0, ctx // unroll, jbody, accs)
                for k in range(nk):
                    hid_v[c * rpc + t,
                          pl.ds(k * _LANES, _LANES)] = accs[k] * inv
        pltpu.sync_copy(hid_v, out_hbm.at[pl.ds(wid * bpw, bpw)])

    return body(ids3, table)


def _mm_body(wt_ref, h_ref, b_ref, o_ref):
    wt = wt_ref[...].astype(jnp.bfloat16)
    h = h_ref[...].astype(jnp.bfloat16)
    acc = lax.dot_general(wt, h, (((0,), (1,)), ((), ())),
                          preferred_element_type=jnp.float32)
    bias = jnp.transpose(b_ref[...], (1, 0))
    o_ref[...] = acc + bias


def _tc_matmul_t(hidden, Wt, b2, vblk):
    batch, dim = hidden.shape
    vocab = Wt.shape[1]
    grid = (pl.cdiv(vocab, vblk),)
    return pl.pallas_call(
        _mm_body,
        grid=grid,
        in_specs=[
            pl.BlockSpec((dim, vblk), lambda i: (0, i)),
            pl.BlockSpec((batch, dim), lambda i: (0, 0)),
            pl.BlockSpec((1, vblk), lambda i: (0, i)),
        ],
        out_specs=pl.BlockSpec((vblk, batch), lambda i: (i, 0)),
        out_shape=jax.ShapeDtypeStruct((vocab, batch), jnp.float32),
        compiler_params=pltpu.CompilerParams(
            dimension_semantics=("parallel",)),
    )(Wt, hidden, b2)


def kernel(context_ids, emb_table, W, b):
    batch, ctx = context_ids.shape
    vocab, dim = emb_table.shape

    bpw = batch // _NW
    ids3 = context_ids.astype(jnp.int32).reshape(_NW, bpw // 2, 2 * ctx)

    hidden = _sc_mean_pool(ids3, emb_table, batch, ctx, dim)
    out_t = _tc_matmul_t(hidden, W.T, b.reshape(1, vocab), 2048)
    return out_t.T
